# baseline (speedup 1.0000x reference)
import functools

import jax
import jax.numpy as jnp
from jax import lax
from jax.experimental import pallas as pl
from jax.experimental.pallas import tpu as pltpu
from jax.experimental.pallas import tpu_sc as plsc

D = 64
S = 65
V = 38
B = 16384
NC, NS, L = 2, 16, 16
NW = NC * NS
BOARDS_PER_W = B // NW
GB = 16
GROUPS = BOARDS_PER_W // GB
BPH = GB // 2
FR = S * V
SPT = 5


def _body(inputs_hbm, token_hbm, pos_hbm, out_hbm,
          token_v, pos_v, fused_s, spmem, offs, idx,
          rows0, rows1, gsem, osem0, osem1):
    cid = lax.axis_index("c")
    sid = lax.axis_index("s")
    wid = sid * NC + cid
    tid = sid

    pltpu.sync_copy(token_hbm, token_v)
    pltpu.sync_copy(pos_hbm, pos_v)
    for k in range(SPT):
        s = lax.min(tid * SPT + k, S - 1)
        pos_row = [pos_v[pl.ds(s * D + j * L, L)] for j in range(D // L)]
        for v in range(V):
            for j in range(D // L):
                fused_s[v, pl.ds(j * L, L)] = (
                    token_v[pl.ds(v * D + j * L, L)] + pos_row[j])
        pltpu.sync_copy(fused_s, spmem.at[pl.ds(s * V, V)])
    plsc.subcore_barrier()

    for k in range(4):
        p = lax.broadcasted_iota(jnp.int32, (L,), 0) + (k * L)
        offs[pl.ds(k * L, L)] = p * V
    lane = lax.broadcasted_iota(jnp.int32, (L,), 0)
    offs[pl.ds(4 * L, L)] = jnp.where(lane == L - 1, (S - 1) * V, 0)

    board_base = wid * BOARDS_PER_W
    bufs = ((rows0, osem0), (rows1, osem1))

    def emit_group(g, first):
        b0 = board_base + g * GB
        pltpu.sync_copy(inputs_hbm.at[pl.ds(b0, GB)], idx)
        for r in range(GB):
            for k in range(4):
                sl = pl.ds(k * L, L)
                idx[r, sl] = idx[r, sl] + offs[sl]
            tl = pl.ds(S - L, L)
            idx[r, tl] = idx[r, tl] + offs[pl.ds(4 * L, L)]
        for half, (buf, osem) in enumerate(bufs):
            if not first:
                pltpu.make_async_copy(
                    buf, out_hbm.at[pl.ds(0, BPH)], osem).wait()
            cps = [
                pltpu.async_copy(
                    spmem.at[idx.at[half * BPH + r]], buf.at[r], gsem)
                for r in range(BPH)
            ]
            for c in cps:
                c.wait()
            pltpu.async_copy(
                buf, out_hbm.at[pl.ds(b0 + half * BPH, BPH)], osem)

    emit_group(0, True)
    lax.fori_loop(1, GROUPS, lambda g, c: (emit_group(g, False), c)[1], 0)
    for buf, osem in bufs:
        pltpu.make_async_copy(buf, out_hbm.at[pl.ds(0, BPH)], osem).wait()


@jax.jit
def kernel(inputs, token_table, pos_table):
    mesh = plsc.VectorSubcoreMesh(
        core_axis_name="c", subcore_axis_name="s",
        num_cores=NC, num_subcores=NS)
    run = functools.partial(
        pl.kernel,
        out_type=jax.ShapeDtypeStruct((B, S, D), jnp.float32),
        mesh=mesh,
        scratch_types=[
            pltpu.VMEM((V * D,), jnp.float32),
            pltpu.VMEM((S * D,), jnp.float32),
            pltpu.VMEM((V, D), jnp.float32),
            pltpu.VMEM_SHARED((FR, D), jnp.float32),
            pltpu.VMEM((5 * L,), jnp.int32),
            pltpu.VMEM((GB, S), jnp.int32),
            pltpu.VMEM((BPH, S, D), jnp.float32),
            pltpu.VMEM((BPH, S, D), jnp.float32),
            pltpu.SemaphoreType.DMA,
            pltpu.SemaphoreType.DMA,
            pltpu.SemaphoreType.DMA,
        ],
        compiler_params=pltpu.CompilerParams(use_tc_tiling_on_sc=False),
    )(_body)
    return run(inputs, token_table.reshape(V * D), pos_table.reshape(S * D))
